# baseline (speedup 1.0000x reference)
import jax
import jax.numpy as jnp
from jax.experimental import pallas as pl
from jax.experimental.pallas import tpu as pltpu
from jax.experimental.pallas import tpu_sc as plsc

_NUM_TOKENS = 8192
_TOKEN_DIM = 256
_N = 18432

_BM = 256
_GATHER_WINDOW = 128


def _argmin_body(x_ref, et_ref, idx_ref):
    x = x_ref[...]
    et = et_ref[...]
    dot = jax.lax.dot_general(
        x, et, (((1,), (0,)), ((), ())), preferred_element_type=jnp.float32
    )
    x2 = jnp.sum(x * x, axis=1, keepdims=True)
    e2 = jnp.sum(et * et, axis=0, keepdims=True)
    dist = (x2 + e2) - 2.0 * dot
    m = jnp.min(dist, axis=1, keepdims=True)
    iota = jax.lax.broadcasted_iota(jnp.int32, dist.shape, 1)
    idx_ref[...] = jnp.min(jnp.where(dist == m, iota, _NUM_TOKENS), axis=1)


def _compute_indices(x, embed_t):
    return pl.pallas_call(
        _argmin_body,
        grid=(_N // _BM,),
        in_specs=[
            pl.BlockSpec((_BM, _TOKEN_DIM), lambda i: (i, 0)),
            pl.BlockSpec((_TOKEN_DIM, _NUM_TOKENS), lambda i: (0, 0)),
        ],
        out_specs=pl.BlockSpec((_BM,), lambda i: (i,)),
        out_shape=jax.ShapeDtypeStruct((_N,), jnp.int32),
    )(x, embed_t)


def _sc_gather(embed, indices):
    indices = indices.reshape((1, _N))

    @pl.kernel(
        out_type=jax.ShapeDtypeStruct((_N, _TOKEN_DIM), embed.dtype),
        mesh=plsc.VectorSubcoreMesh(
            core_axis_name="core", subcore_axis_name="subcore"
        ),
    )
    def gather_kernel(e_hbm, i_hbm, o_hbm):
        def body(i_vmem, o_vmem):
            pltpu.sync_copy(e_hbm.at[i_vmem.at[0]], o_vmem)

        pltpu.emit_pipeline(
            body,
            grid=(_N // _GATHER_WINDOW,),
            in_specs=[
                pl.BlockSpec((1, _GATHER_WINDOW), index_map=lambda i: (0, i))
            ],
            out_specs=[
                pl.BlockSpec(
                    (_GATHER_WINDOW, _TOKEN_DIM), index_map=lambda i: (i, 0)
                )
            ],
            core_axis_name="subcore",
            dimension_semantics=(pltpu.PARALLEL,),
        )(i_hbm, o_hbm)

    return gather_kernel(embed, indices)


def kernel(inputs_flatten, embed):
    embed_t = embed.T
    idx = _compute_indices(inputs_flatten, embed_t)
    quantize = _sc_gather(embed, idx)
    return (quantize, idx[:, None])


# speedup vs baseline: 1.3298x; 1.3298x over previous
import jax
import jax.numpy as jnp
from jax.experimental import pallas as pl
from jax.experimental.pallas import tpu as pltpu
from jax.experimental.pallas import tpu_sc as plsc

_NUM_TOKENS = 8192
_TOKEN_DIM = 256
_N = 18432

_BM = 256
_GATHER_WINDOW = 128


def _argmin_body(x_ref, m2et_ref, idx_ref, aux_ref):
    @pl.when(pl.program_id(0) == 0)
    def _init():
        m2et = m2et_ref[...]
        aux_ref[0:1, :] = 0.25 * jnp.sum(m2et * m2et, axis=0, keepdims=True)
        aux_ref[1:2, :] = jax.lax.broadcasted_iota(
            jnp.int32, (1, _NUM_TOKENS), 1
        ).astype(jnp.float32)

    x = x_ref[...]
    x2 = jnp.sum(x * x, axis=1, keepdims=True)
    dotm2 = jax.lax.dot_general(
        x, m2et_ref[...], (((1,), (0,)), ((), ())),
        preferred_element_type=jnp.float32,
    )
    dist = (x2 + aux_ref[0:1, :]) + dotm2
    m = jnp.min(dist, axis=1, keepdims=True)
    idxf = jnp.min(
        jnp.where(dist == m, aux_ref[1:2, :], jnp.float32(1e9)), axis=1
    )
    idx_ref[...] = idxf.astype(jnp.int32)


def _compute_indices(x, m2embed_t):
    return pl.pallas_call(
        _argmin_body,
        grid=(_N // _BM,),
        in_specs=[
            pl.BlockSpec((_BM, _TOKEN_DIM), lambda i: (i, 0)),
            pl.BlockSpec((_TOKEN_DIM, _NUM_TOKENS), lambda i: (0, 0)),
        ],
        out_specs=pl.BlockSpec((_BM,), lambda i: (i,)),
        out_shape=jax.ShapeDtypeStruct((_N,), jnp.int32),
        scratch_shapes=[pltpu.VMEM((8, _NUM_TOKENS), jnp.float32)],
    )(x, m2embed_t)


def _sc_gather(embed, indices):
    indices = indices.reshape((1, _N))

    @pl.kernel(
        out_type=jax.ShapeDtypeStruct((_N, _TOKEN_DIM), embed.dtype),
        mesh=plsc.VectorSubcoreMesh(
            core_axis_name="core", subcore_axis_name="subcore"
        ),
    )
    def gather_kernel(e_hbm, i_hbm, o_hbm):
        def body(i_vmem, o_vmem):
            pltpu.sync_copy(e_hbm.at[i_vmem.at[0]], o_vmem)

        pltpu.emit_pipeline(
            body,
            grid=(_N // _GATHER_WINDOW,),
            in_specs=[
                pl.BlockSpec((1, _GATHER_WINDOW), index_map=lambda i: (0, i))
            ],
            out_specs=[
                pl.BlockSpec(
                    (_GATHER_WINDOW, _TOKEN_DIM), index_map=lambda i: (i, 0)
                )
            ],
            core_axis_name=("core", "subcore"),
            dimension_semantics=(pltpu.PARALLEL,),
        )(i_hbm, o_hbm)

    return gather_kernel(embed, indices)


def kernel(inputs_flatten, embed):
    m2embed_t = -2.0 * embed.T
    idx = _compute_indices(inputs_flatten, m2embed_t)
    quantize = _sc_gather(embed, idx)
    return (quantize, idx[:, None])


# speedup vs baseline: 1.3323x; 1.0019x over previous
import jax
import jax.numpy as jnp
from jax.experimental import pallas as pl
from jax.experimental.pallas import tpu as pltpu
from jax.experimental.pallas import tpu_sc as plsc

_NUM_TOKENS = 8192
_TOKEN_DIM = 256
_N = 18432

_BM = 512
_GATHER_WINDOW = 128


def _argmin_body(x_ref, m2et_ref, idx_ref, aux_ref):
    @pl.when(pl.program_id(0) == 0)
    def _init():
        m2et = m2et_ref[...]
        aux_ref[0:1, :] = 0.25 * jnp.sum(m2et * m2et, axis=0, keepdims=True)
        aux_ref[1:2, :] = jax.lax.broadcasted_iota(
            jnp.int32, (1, _NUM_TOKENS), 1
        ).astype(jnp.float32)

    x = x_ref[...]
    x2 = jnp.sum(x * x, axis=1, keepdims=True)
    dotm2 = jax.lax.dot_general(
        x, m2et_ref[...], (((1,), (0,)), ((), ())),
        preferred_element_type=jnp.float32,
    )
    dist = (x2 + aux_ref[0:1, :]) + dotm2
    m = jnp.min(dist, axis=1, keepdims=True)
    idxf = jnp.min(
        jnp.where(dist == m, aux_ref[1:2, :], jnp.float32(1e9)), axis=1
    )
    idx_ref[...] = idxf.astype(jnp.int32)


def _compute_indices(x, m2embed_t):
    return pl.pallas_call(
        _argmin_body,
        grid=(_N // _BM,),
        in_specs=[
            pl.BlockSpec((_BM, _TOKEN_DIM), lambda i: (i, 0)),
            pl.BlockSpec((_TOKEN_DIM, _NUM_TOKENS), lambda i: (0, 0)),
        ],
        out_specs=pl.BlockSpec((_BM,), lambda i: (i,)),
        out_shape=jax.ShapeDtypeStruct((_N,), jnp.int32),
        scratch_shapes=[pltpu.VMEM((8, _NUM_TOKENS), jnp.float32)],
    )(x, m2embed_t)


def _sc_gather(embed, indices):
    indices = indices.reshape((1, _N))

    @pl.kernel(
        out_type=jax.ShapeDtypeStruct((_N, _TOKEN_DIM), embed.dtype),
        mesh=plsc.VectorSubcoreMesh(
            core_axis_name="core", subcore_axis_name="subcore"
        ),
    )
    def gather_kernel(e_hbm, i_hbm, o_hbm):
        def body(i_vmem, o_vmem):
            pltpu.sync_copy(e_hbm.at[i_vmem.at[0]], o_vmem)

        pltpu.emit_pipeline(
            body,
            grid=(_N // _GATHER_WINDOW,),
            in_specs=[
                pl.BlockSpec((1, _GATHER_WINDOW), index_map=lambda i: (0, i))
            ],
            out_specs=[
                pl.BlockSpec(
                    (_GATHER_WINDOW, _TOKEN_DIM), index_map=lambda i: (i, 0)
                )
            ],
            core_axis_name=("core", "subcore"),
            dimension_semantics=(pltpu.PARALLEL,),
        )(i_hbm, o_hbm)

    return gather_kernel(embed, indices)


def kernel(inputs_flatten, embed):
    m2embed_t = -2.0 * embed.T
    idx = _compute_indices(inputs_flatten, m2embed_t)
    quantize = _sc_gather(embed, idx)
    return (quantize, idx[:, None])
